# baseline (speedup 1.0000x reference)
import functools

import jax
import jax.numpy as jnp
from jax import lax
from jax.experimental import pallas as pl
from jax.experimental.pallas import tpu as pltpu
from jax.experimental.pallas import tpu_sc as plsc

N = 10000
E = 320000
D_IN = 128
H = 16
C = 2

NC = 2
NS = 16
NW = NC * NS
CH = 128
K = 80
EPT = K * CH
EPAD = NW * EPT
RPT = N // NS
NB = 8

_mesh = plsc.VectorSubcoreMesh(core_axis_name="c", subcore_axis_name="s")
_sc_params = pltpu.CompilerParams(use_tc_tiling_on_sc=False,
                                  needs_layout_passes=False)


def _newton_rsqrt(d):
    i = plsc.bitcast(d, jnp.int32)
    i = jnp.int32(0x5F3759DF) - lax.shift_right_logical(i, 1)
    y = plsc.bitcast(i, jnp.float32)
    hd = 0.5 * d
    for _ in range(3):
        y = y * (1.5 - hd * y * y)
    return y


def _zero_1008(buf):
    def body(i, _):
        buf[pl.ds(i * 16, 16)] = jnp.zeros((16,), jnp.float32)
        return 0

    lax.fori_loop(0, 63, body, 0)


def _zero_rows(buf, n):
    def body(i, _):
        buf[i, :] = jnp.zeros((H,), jnp.float32)
        return 0

    lax.fori_loop(0, n, body, 0)


def _mp_phase(table_hbm, src_v, dst_v, ew_v, dis_v, rows_v, acc_sh,
              gsems, ssem):
    def gwait(b):
        pltpu.make_async_copy(
            table_hbm.at[pl.ds(0, CH)], rows_v.at[b], gsems[b]).wait()

    def swait(b):
        pltpu.make_async_copy(
            table_hbm.at[pl.ds(0, CH)], rows_v.at[b], ssem).wait()

    def gstart(j, b):
        pltpu.async_copy(table_hbm.at[pl.ds(0, CH)], rows_v.at[b], gsems[b])

    for b in range(NB // 2):
        gstart(b, b)

    def outer(t, _):
        for b in range(NB):
            j = NB * t + b
            gwait(b)

            @plsc.parallel_loop(0, 0, unroll=2)
            def scale(g):
                base = g * 16
                srcv = src_v[j, pl.ds(base, 16)]
                s16 = ew_v[j, pl.ds(base, 16)] * plsc.load_gather(dis_v, [srcv])
                for t16 in range(16):
                    rows_v[b, base + t16, :] = rows_v[b, base + t16, :] * s16[t16]

            @pl.when(j < 0)
            def _():
                swait(b)

            @pl.when(j < 0)
            def _():
                pltpu.async_copy(rows_v.at[b], acc_sh.at[dst_v.at[j]], ssem,
                                 add=True)

            @pl.when(j + NB // 2 < K)
            def _():
                gstart(j + NB // 2, (b + NB // 2) % NB)
        return 0

    lax.fori_loop(0, K // NB, outer, 0)
    for b in range(0):
        swait(b)


@functools.partial(
    pl.kernel,
    out_type=(
        jax.ShapeDtypeStruct((NC, N, H), jnp.float32),
        jax.ShapeDtypeStruct((N,), jnp.float32),
    ),
    mesh=_mesh,
    scratch_types=[
        pltpu.VMEM((K, CH), jnp.int32),
        pltpu.VMEM((2, K, CH), jnp.int32),
        pltpu.VMEM((2, K, CH), jnp.float32),
        pltpu.VMEM((N,), jnp.float32),
        pltpu.VMEM((1008,), jnp.float32),
        pltpu.VMEM((NB, CH, H), jnp.float32),
        pltpu.VMEM((RPT, H), jnp.float32),
        pltpu.VMEM_SHARED((N,), jnp.float32),
        pltpu.VMEM_SHARED((N, H), jnp.float32),
        [pltpu.SemaphoreType.DMA] * NB,
        pltpu.SemaphoreType.DMA,
        pltpu.SemaphoreType.DMA,
    ],
    compiler_params=_sc_params,
)
def _k_mp1(table_hbm, src_hbm, dst_hbm, ew_hbm, acc_out, dis_out,
           src_v, dstd_v, ewd_v, dis_v, zbuf1, rows_v, zbuf,
           deg_sh, acc_sh, gsems, ssem, dsem):
    cid = lax.axis_index("c")
    sid = lax.axis_index("s")
    wid = cid * NS + sid

    cps = [
        pltpu.async_copy(src_hbm.at[wid], src_v, dsem),
        pltpu.async_copy(dst_hbm.at[sid], dstd_v.at[0], dsem),
        pltpu.async_copy(dst_hbm.at[NS + sid], dstd_v.at[1], dsem),
        pltpu.async_copy(ew_hbm.at[sid], ewd_v.at[0], dsem),
        pltpu.async_copy(ew_hbm.at[NS + sid], ewd_v.at[1], dsem),
    ]
    _zero_1008(zbuf1)
    _zero_rows(zbuf, RPT)
    for cp in cps:
        cp.wait()

    @pl.when(sid < 10)
    def _():
        pltpu.sync_copy(zbuf1.at[pl.ds(0, 1000)],
                        deg_sh.at[pl.ds(sid * 1000, 1000)])

    pltpu.sync_copy(zbuf, acc_sh.at[pl.ds(sid * RPT, RPT)])

    plsc.subcore_barrier()

    def deg_body(j, _):
        g = j // K
        r = j - g * K
        pltpu.async_copy(ewd_v.at[g, r], deg_sh.at[dstd_v.at[g, r]], dsem,
                         add=True)

        @pl.when(j >= NB - 1)
        def _():
            pltpu.make_async_copy(ew_hbm.at[0, 0], zbuf1.at[pl.ds(0, CH)],
                                  dsem).wait()
        return 0

    lax.fori_loop(0, 2 * K, deg_body, 0)
    for _ in range(NB - 1):
        pltpu.make_async_copy(ew_hbm.at[0, 0], zbuf1.at[pl.ds(0, CH)],
                              dsem).wait()

    plsc.subcore_barrier()

    pltpu.sync_copy(deg_sh, dis_v)

    @plsc.parallel_loop(0, N // 16, unroll=2)
    def dis_body(i):
        d = dis_v[pl.ds(i * 16, 16)] + 1.0
        dis_v[pl.ds(i * 16, 16)] = _newton_rsqrt(d)

    @pl.when((cid == 0) & (sid < 10))
    def _():
        pltpu.sync_copy(dis_v.at[pl.ds(sid * 1000, 1000)],
                        dis_out.at[pl.ds(sid * 1000, 1000)])

    _mp_phase(table_hbm, src_v, dstd_v.at[cid], ewd_v.at[cid], dis_v,
              rows_v, acc_sh, gsems, ssem)

    plsc.subcore_barrier()
    pltpu.sync_copy(acc_sh.at[pl.ds(sid * RPT, RPT)], zbuf)
    pltpu.sync_copy(zbuf, acc_out.at[cid, pl.ds(sid * RPT, RPT)])


@functools.partial(
    pl.kernel,
    out_type=(
        jax.ShapeDtypeStruct((N, H), jnp.float32),
        jax.ShapeDtypeStruct((NC, N, H), jnp.float32),
    ),
    mesh=_mesh,
    scratch_types=[
        pltpu.VMEM((K, CH), jnp.int32),
        pltpu.VMEM((K, CH), jnp.int32),
        pltpu.VMEM((K, CH), jnp.float32),
        pltpu.VMEM((N,), jnp.float32),
        pltpu.VMEM((NB, CH, H), jnp.float32),
        pltpu.VMEM((RPT, H), jnp.float32),
        pltpu.VMEM((RPT, H), jnp.float32),
        pltpu.VMEM((RPT, H), jnp.float32),
        pltpu.VMEM((RPT, H), jnp.float32),
        pltpu.VMEM((RPT, H), jnp.float32),
        pltpu.VMEM((16,), jnp.float32),
        pltpu.VMEM_SHARED((N, H), jnp.float32),
        [pltpu.SemaphoreType.DMA] * NB,
        pltpu.SemaphoreType.DMA,
    ],
    compiler_params=_sc_params,
)
def _k_mp2(acc1_hbm, xw_hbm, dis_hbm, b1_hbm, src_hbm, dst_hbm, ew_hbm,
           h_out, acc_out,
           src_v, dst_v, ew_v, dis_v, rows_v, zbuf, a0_v, a1_v, xw_v, h_v,
           b1_v, acc_sh, gsems, ssem):
    cid = lax.axis_index("c")
    sid = lax.axis_index("s")
    wid = cid * NS + sid
    row0 = sid * RPT

    cps = [
        pltpu.async_copy(src_hbm.at[wid], src_v, ssem),
        pltpu.async_copy(dst_hbm.at[wid], dst_v, ssem),
        pltpu.async_copy(ew_hbm.at[wid], ew_v, ssem),
        pltpu.async_copy(dis_hbm, dis_v, ssem),
        pltpu.async_copy(b1_hbm, b1_v, ssem),
        pltpu.async_copy(acc1_hbm.at[0, pl.ds(row0, RPT)], a0_v, ssem),
        pltpu.async_copy(acc1_hbm.at[1, pl.ds(row0, RPT)], a1_v, ssem),
        pltpu.async_copy(xw_hbm.at[pl.ds(row0, RPT)], xw_v, ssem),
    ]
    _zero_rows(zbuf, RPT)
    for cp in cps:
        cp.wait()

    b1v = b1_v[...]

    @plsc.parallel_loop(0, RPT // 16, unroll=2)
    def h_body(q):
        dis16 = dis_v[pl.ds(row0 + q * 16, 16)]
        for t16 in range(16):
            r = q * 16 + t16
            d = dis16[t16]
            s = a0_v[r, :] + a1_v[r, :] + d * xw_v[r, :]
            h_v[r, :] = jnp.maximum(d * s + b1v, 0.0)
    q625 = RPT - 1
    dlast = dis_v[pl.ds(row0 + q625 - 15, 16)]
    slast = a0_v[q625, :] + a1_v[q625, :] + dlast[15] * xw_v[q625, :]
    h_v[q625, :] = jnp.maximum(dlast[15] * slast + b1v, 0.0)

    pltpu.sync_copy(h_v, h_out.at[pl.ds(row0, RPT)])
    pltpu.sync_copy(zbuf, acc_sh.at[pl.ds(row0, RPT)])

    plsc.subcore_barrier()

    _mp_phase(h_out, src_v, dst_v, ew_v, dis_v, rows_v, acc_sh, gsems, ssem)

    plsc.subcore_barrier()
    pltpu.sync_copy(acc_sh.at[pl.ds(row0, RPT)], zbuf)
    pltpu.sync_copy(zbuf, acc_out.at[cid, pl.ds(row0, RPT)])


def _lin1_body(x_ref, w_ref, table_ref):
    table_ref[...] = jnp.dot(x_ref[...], w_ref[...],
                             preferred_element_type=jnp.float32)


_k_lin1 = pl.pallas_call(
    _lin1_body,
    out_shape=jax.ShapeDtypeStruct((N, H), jnp.float32),
)


def _out_body(accp_ref, h_ref, dis_ref, w2_ref, b2_ref, o_ref):
    dis = dis_ref[...]
    s = accp_ref[0] + accp_ref[1] + dis * h_ref[...]
    o_ref[...] = (
        jnp.dot(dis * s, w2_ref[...], preferred_element_type=jnp.float32)
        + b2_ref[...]
    )


_k_out = pl.pallas_call(
    _out_body,
    out_shape=jax.ShapeDtypeStruct((N, C), jnp.float32),
)


def kernel(x, edge_index, edge_weight, W1, b1, W2, b2):
    src = edge_index[0]
    dst = edge_index[1]
    pad = EPAD - E
    srcp = jnp.concatenate([src, jnp.zeros((pad,), src.dtype)]).reshape(NW, K, CH)
    dstp = jnp.concatenate([dst, jnp.zeros((pad,), dst.dtype)]).reshape(NW, K, CH)
    ewp = jnp.concatenate(
        [edge_weight, jnp.zeros((pad,), edge_weight.dtype)]
    ).reshape(NW, K, CH)

    table1 = _k_lin1(x, W1)
    acc1, dis = _k_mp1(table1, srcp, dstp, ewp)
    h, acc2 = _k_mp2(acc1, table1, dis, b1, srcp, dstp, ewp)
    out = _k_out(acc2, h, dis.reshape(N, 1), W2, b2.reshape(1, C))
    return (h, out)


# speedup vs baseline: 2.0602x; 2.0602x over previous
import functools

import jax
import jax.numpy as jnp
from jax import lax
from jax.experimental import pallas as pl
from jax.experimental.pallas import tpu as pltpu
from jax.experimental.pallas import tpu_sc as plsc

N = 10000
E = 320000
D_IN = 128
H = 16
C = 2

NC = 2
NS = 16
NW = NC * NS
CH = 128
K = 80
EPT = K * CH
EPAD = NW * EPT
RPT = N // NS
NB = 8

_mesh = plsc.VectorSubcoreMesh(core_axis_name="c", subcore_axis_name="s")
_sc_params = pltpu.CompilerParams(use_tc_tiling_on_sc=False,
                                  needs_layout_passes=False)


def _newton_rsqrt(d):
    i = plsc.bitcast(d, jnp.int32)
    i = jnp.int32(0x5F3759DF) - lax.shift_right_logical(i, 1)
    y = plsc.bitcast(i, jnp.float32)
    hd = 0.5 * d
    for _ in range(3):
        y = y * (1.5 - hd * y * y)
    return y


def _zero_1008(buf):
    def body(i, _):
        buf[pl.ds(i * 16, 16)] = jnp.zeros((16,), jnp.float32)
        return 0

    lax.fori_loop(0, 63, body, 0)


def _zero_rows(buf, n):
    def body(i, _):
        buf[i, :] = jnp.zeros((H,), jnp.float32)
        return 0

    lax.fori_loop(0, n, body, 0)


def _mp_phase(tbl_sh, drain_hbm, src_v, dst_v, ew_v, dis_v, rows_v, acc_sh,
              gsems, ssem):
    def gwait(b):
        pltpu.make_async_copy(
            drain_hbm.at[pl.ds(0, CH)], rows_v.at[b], gsems[b]).wait()

    def swait(b):
        pltpu.make_async_copy(
            drain_hbm.at[pl.ds(0, CH)], rows_v.at[b], ssem).wait()

    def gstart(j, b):
        pltpu.async_copy(tbl_sh.at[src_v.at[j]], rows_v.at[b], gsems[b])

    for b in range(NB // 2):
        gstart(b, b)

    def outer(t, _):
        for b in range(NB):
            j = NB * t + b
            gwait(b)

            @plsc.parallel_loop(0, CH // 16, unroll=2)
            def scale(g):
                base = g * 16
                srcv = src_v[j, pl.ds(base, 16)]
                s16 = ew_v[j, pl.ds(base, 16)] * plsc.load_gather(dis_v, [srcv])
                for t16 in range(16):
                    rows_v[b, base + t16, :] = rows_v[b, base + t16, :] * s16[t16]

            @pl.when(j >= NB // 2)
            def _():
                swait(b)

            pltpu.async_copy(rows_v.at[b], acc_sh.at[dst_v.at[j]], ssem,
                             add=True)

            @pl.when(j + NB // 2 < K)
            def _():
                gstart(j + NB // 2, (b + NB // 2) % NB)
        return 0

    lax.fori_loop(0, K // NB, outer, 0)
    for b in range(NB // 2):
        swait(b)


@functools.partial(
    pl.kernel,
    out_type=(
        jax.ShapeDtypeStruct((NC, N, H), jnp.float32),
        jax.ShapeDtypeStruct((N,), jnp.float32),
    ),
    mesh=_mesh,
    scratch_types=[
        pltpu.VMEM((K, CH), jnp.int32),
        pltpu.VMEM((2, K, CH), jnp.int32),
        pltpu.VMEM((2, K, CH), jnp.float32),
        pltpu.VMEM((N,), jnp.float32),
        pltpu.VMEM((1008,), jnp.float32),
        pltpu.VMEM((NB, CH, H), jnp.float32),
        pltpu.VMEM((RPT, H), jnp.float32),
        pltpu.VMEM_SHARED((N,), jnp.float32),
        pltpu.VMEM_SHARED((N, H), jnp.float32),
        pltpu.VMEM_SHARED((N, H), jnp.float32),
        [pltpu.SemaphoreType.DMA] * NB,
        pltpu.SemaphoreType.DMA,
        pltpu.SemaphoreType.DMA,
    ],
    compiler_params=_sc_params,
)
def _k_mp1(table_hbm, src_hbm, dst_hbm, ew_hbm, acc_out, dis_out,
           src_v, dstd_v, ewd_v, dis_v, zbuf1, rows_v, zbuf,
           deg_sh, acc_sh, tbl_sh, gsems, ssem, dsem):
    cid = lax.axis_index("c")
    sid = lax.axis_index("s")
    wid = cid * NS + sid
    row0 = sid * RPT

    cps = [
        pltpu.async_copy(src_hbm.at[wid], src_v, dsem),
        pltpu.async_copy(dst_hbm.at[sid], dstd_v.at[0], dsem),
        pltpu.async_copy(dst_hbm.at[NS + sid], dstd_v.at[1], dsem),
        pltpu.async_copy(ew_hbm.at[sid], ewd_v.at[0], dsem),
        pltpu.async_copy(ew_hbm.at[NS + sid], ewd_v.at[1], dsem),
        pltpu.async_copy(table_hbm.at[pl.ds(row0, RPT)],
                         tbl_sh.at[pl.ds(row0, RPT)], dsem),
    ]
    _zero_1008(zbuf1)
    _zero_rows(zbuf, RPT)
    for cp in cps:
        cp.wait()

    @pl.when(sid < 10)
    def _():
        pltpu.sync_copy(zbuf1.at[pl.ds(0, 1000)],
                        deg_sh.at[pl.ds(sid * 1000, 1000)])

    pltpu.sync_copy(zbuf, acc_sh.at[pl.ds(row0, RPT)])

    plsc.subcore_barrier()

    def deg_body(j, _):
        g = j // K
        r = j - g * K
        pltpu.async_copy(ewd_v.at[g, r], deg_sh.at[dstd_v.at[g, r]], dsem,
                         add=True)

        @pl.when(j >= NB - 1)
        def _():
            pltpu.make_async_copy(ew_hbm.at[0, 0], zbuf1.at[pl.ds(0, CH)],
                                  dsem).wait()
        return 0

    lax.fori_loop(0, 2 * K, deg_body, 0)
    for _ in range(NB - 1):
        pltpu.make_async_copy(ew_hbm.at[0, 0], zbuf1.at[pl.ds(0, CH)],
                              dsem).wait()

    plsc.subcore_barrier()

    pltpu.sync_copy(deg_sh, dis_v)

    @plsc.parallel_loop(0, N // 16, unroll=2)
    def dis_body(i):
        d = dis_v[pl.ds(i * 16, 16)] + 1.0
        dis_v[pl.ds(i * 16, 16)] = _newton_rsqrt(d)

    @pl.when((cid == 0) & (sid < 10))
    def _():
        pltpu.sync_copy(dis_v.at[pl.ds(sid * 1000, 1000)],
                        dis_out.at[pl.ds(sid * 1000, 1000)])

    _mp_phase(tbl_sh, table_hbm, src_v, dstd_v.at[cid], ewd_v.at[cid], dis_v,
              rows_v, acc_sh, gsems, ssem)

    plsc.subcore_barrier()
    pltpu.sync_copy(acc_sh.at[pl.ds(row0, RPT)], zbuf)
    pltpu.sync_copy(zbuf, acc_out.at[cid, pl.ds(row0, RPT)])


@functools.partial(
    pl.kernel,
    out_type=(
        jax.ShapeDtypeStruct((N, H), jnp.float32),
        jax.ShapeDtypeStruct((NC, N, H), jnp.float32),
    ),
    mesh=_mesh,
    scratch_types=[
        pltpu.VMEM((K, CH), jnp.int32),
        pltpu.VMEM((K, CH), jnp.int32),
        pltpu.VMEM((K, CH), jnp.float32),
        pltpu.VMEM((N,), jnp.float32),
        pltpu.VMEM((NB, CH, H), jnp.float32),
        pltpu.VMEM((RPT, H), jnp.float32),
        pltpu.VMEM((RPT, H), jnp.float32),
        pltpu.VMEM((RPT, H), jnp.float32),
        pltpu.VMEM((RPT, H), jnp.float32),
        pltpu.VMEM((RPT, H), jnp.float32),
        pltpu.VMEM((16,), jnp.float32),
        pltpu.VMEM_SHARED((N, H), jnp.float32),
        pltpu.VMEM_SHARED((N, H), jnp.float32),
        [pltpu.SemaphoreType.DMA] * NB,
        pltpu.SemaphoreType.DMA,
    ],
    compiler_params=_sc_params,
)
def _k_mp2(acc1_hbm, xw_hbm, dis_hbm, b1_hbm, src_hbm, dst_hbm, ew_hbm,
           h_out, acc_out,
           src_v, dst_v, ew_v, dis_v, rows_v, zbuf, a0_v, a1_v, xw_v, h_v,
           b1_v, acc_sh, tbl_sh, gsems, ssem):
    cid = lax.axis_index("c")
    sid = lax.axis_index("s")
    wid = cid * NS + sid
    row0 = sid * RPT

    cps = [
        pltpu.async_copy(src_hbm.at[wid], src_v, ssem),
        pltpu.async_copy(dst_hbm.at[wid], dst_v, ssem),
        pltpu.async_copy(ew_hbm.at[wid], ew_v, ssem),
        pltpu.async_copy(dis_hbm, dis_v, ssem),
        pltpu.async_copy(b1_hbm, b1_v, ssem),
        pltpu.async_copy(acc1_hbm.at[0, pl.ds(row0, RPT)], a0_v, ssem),
        pltpu.async_copy(acc1_hbm.at[1, pl.ds(row0, RPT)], a1_v, ssem),
        pltpu.async_copy(xw_hbm.at[pl.ds(row0, RPT)], xw_v, ssem),
    ]
    _zero_rows(zbuf, RPT)
    for cp in cps:
        cp.wait()

    b1v = b1_v[...]

    @plsc.parallel_loop(0, RPT // 16, unroll=2)
    def h_body(q):
        dis16 = dis_v[pl.ds(row0 + q * 16, 16)]
        for t16 in range(16):
            r = q * 16 + t16
            d = dis16[t16]
            s = a0_v[r, :] + a1_v[r, :] + d * xw_v[r, :]
            h_v[r, :] = jnp.maximum(d * s + b1v, 0.0)

    q625 = RPT - 1
    dlast = dis_v[pl.ds(row0 + q625 - 15, 16)]
    slast = a0_v[q625, :] + a1_v[q625, :] + dlast[15] * xw_v[q625, :]
    h_v[q625, :] = jnp.maximum(dlast[15] * slast + b1v, 0.0)

    pltpu.sync_copy(h_v, tbl_sh.at[pl.ds(row0, RPT)])

    @pl.when(cid == 0)
    def _():
        pltpu.sync_copy(h_v, h_out.at[pl.ds(row0, RPT)])

    pltpu.sync_copy(zbuf, acc_sh.at[pl.ds(row0, RPT)])

    plsc.subcore_barrier()

    _mp_phase(tbl_sh, h_out, src_v, dst_v, ew_v, dis_v, rows_v, acc_sh,
              gsems, ssem)

    plsc.subcore_barrier()
    pltpu.sync_copy(acc_sh.at[pl.ds(row0, RPT)], zbuf)
    pltpu.sync_copy(zbuf, acc_out.at[cid, pl.ds(row0, RPT)])


def _lin1_body(x_ref, w_ref, table_ref):
    table_ref[...] = jnp.dot(x_ref[...], w_ref[...],
                             preferred_element_type=jnp.float32)


_k_lin1 = pl.pallas_call(
    _lin1_body,
    out_shape=jax.ShapeDtypeStruct((N, H), jnp.float32),
)


def _out_body(accp_ref, h_ref, dis_ref, w2_ref, b2_ref, o_ref):
    dis = dis_ref[...]
    s = accp_ref[0] + accp_ref[1] + dis * h_ref[...]
    o_ref[...] = (
        jnp.dot(dis * s, w2_ref[...], preferred_element_type=jnp.float32)
        + b2_ref[...]
    )


_k_out = pl.pallas_call(
    _out_body,
    out_shape=jax.ShapeDtypeStruct((N, C), jnp.float32),
)


def kernel(x, edge_index, edge_weight, W1, b1, W2, b2):
    src = edge_index[0]
    dst = edge_index[1]
    pad = EPAD - E
    srcp = jnp.concatenate([src, jnp.zeros((pad,), src.dtype)]).reshape(NW, K, CH)
    dstp = jnp.concatenate([dst, jnp.zeros((pad,), dst.dtype)]).reshape(NW, K, CH)
    ewp = jnp.concatenate(
        [edge_weight, jnp.zeros((pad,), edge_weight.dtype)]
    ).reshape(NW, K, CH)

    table1 = _k_lin1(x, W1)
    acc1, dis = _k_mp1(table1, srcp, dstp, ewp)
    h, acc2 = _k_mp2(acc1, table1, dis, b1, srcp, dstp, ewp)
    out = _k_out(acc2, h, dis.reshape(N, 1), W2, b2.reshape(1, C))
    return (h, out)
